# baseline (speedup 1.0000x reference)
import functools

import jax
import jax.numpy as jnp
import numpy as np
from jax import lax
from jax.experimental import pallas as pl
from jax.experimental.pallas import tpu as pltpu
from jax.experimental.pallas import tpu_sc as plsc

D = 1024
VOCAB = 194
PAIRS = D // 2
NC = 2
NS = 16
NW = NC * NS
B_TOT = 1024 * 200
B_PER_W = B_TOT // NW
C = 32
NCHUNK = B_PER_W // C

_PERM = np.arange(D).reshape(D // 32, 2, 16).transpose(0, 2, 1).reshape(-1)


def _sc_gather(ids_flat, table_pairs):
    mesh = plsc.VectorSubcoreMesh(core_axis_name="c", subcore_axis_name="s")

    @functools.partial(
        pl.kernel,
        mesh=mesh,
        compiler_params=pltpu.CompilerParams(
            use_tc_tiling_on_sc=False, needs_layout_passes=False
        ),
        out_type=jax.ShapeDtypeStruct((B_TOT * D,), jnp.float32),
        scratch_types=[
            pltpu.VMEM((B_PER_W,), jnp.int32),
            pltpu.VMEM((C, PAIRS), jnp.int32),
            pltpu.VMEM((C, PAIRS), jnp.int32),
            pltpu.VMEM((C * D,), jnp.float32),
            pltpu.VMEM((C * D,), jnp.float32),
            pltpu.SemaphoreType.DMA,
            pltpu.SemaphoreType.DMA,
            pltpu.SemaphoreType.DMA,
            pltpu.SemaphoreType.DMA,
        ],
    )
    def k(ids_hbm, tbl_hbm, out_hbm, idx_v, bf0, bf1, ob0, ob1, g0, g1, s0, s1):
        wid = lax.axis_index("s") * NC + lax.axis_index("c")
        base = pl.multiple_of(wid * B_PER_W, 8)
        pltpu.sync_copy(ids_hbm.at[pl.ds(base, B_PER_W)], idx_v)

        bfs = (bf0, bf1)
        obs = (ob0, ob1)
        gsems = (g0, g1)
        ssems = (s0, s1)
        HIMASK = jnp.int32(-65536)

        def g_start(chunk, b):
            off = pl.multiple_of(chunk * C, 8)
            pltpu.async_copy(
                tbl_hbm.at[idx_v.at[pl.ds(off, C)]], bfs[b], gsems[b]
            )

        def g_wait(b):
            pltpu.make_async_copy(
                tbl_hbm.at[pl.ds(0, C)], bfs[b], gsems[b]
            ).wait()

        def s_start(chunk, b):
            flat = pl.multiple_of((base + chunk * C) * D, 8)
            pltpu.async_copy(
                obs[b], out_hbm.at[pl.ds(flat, C * D)], ssems[b]
            )

        def s_wait(b):
            pltpu.make_async_copy(
                out_hbm.at[pl.ds(0, C * D)], obs[b], ssems[b]
            ).wait()

        def widen(b):
            def row_body(r, carry):
                dst = pl.multiple_of(r * D, 8)
                for v in range(PAIRS // 16):
                    x = bfs[b][r, pl.ds(v * 16, 16)]
                    lo = plsc.bitcast(x << 16, jnp.float32)
                    hi = plsc.bitcast(x & HIMASK, jnp.float32)
                    obs[b][pl.ds(dst + v * 32, 16)] = lo
                    obs[b][pl.ds(dst + v * 32 + 16, 16)] = hi
                return carry

            lax.fori_loop(0, C, row_body, 0)

        g_start(0, 0)
        g_start(1, 1)

        def pair(p2, carry):
            for b in range(2):
                chunk = p2 * 2 + b
                g_wait(b)

                @pl.when(chunk >= 2)
                def _():
                    s_wait(b)

                widen(b)
                s_start(chunk, b)

                @pl.when(chunk + 2 < NCHUNK)
                def _():
                    g_start(chunk + 2, b)

            return carry

        lax.fori_loop(0, NCHUNK // 2, pair, 0)
        s_wait(0)
        s_wait(1)

    return k(ids_flat, table_pairs)


def kernel(phoneme_ids, table):
    ids_flat = phoneme_ids.reshape(-1)
    tbl_bf = table[:, _PERM].astype(jnp.bfloat16)
    table_pairs = lax.bitcast_convert_type(
        tbl_bf.reshape(VOCAB, PAIRS, 2), jnp.int32
    )
    out = _sc_gather(ids_flat, table_pairs)
    return out.reshape(phoneme_ids.shape[0], phoneme_ids.shape[1], D)


# speedup vs baseline: 1.1966x; 1.1966x over previous
import functools

import jax
import jax.numpy as jnp
import numpy as np
from jax import lax
from jax.experimental import pallas as pl
from jax.experimental.pallas import tpu as pltpu
from jax.experimental.pallas import tpu_sc as plsc

D = 1024
VOCAB = 194
PAIRS = D // 2
NC = 2
NS = 16
NW = NC * NS
B_TOT = 1024 * 200
B_PER_W = B_TOT // NW
C = 32
NCHUNK = B_PER_W // C

_PERM = np.arange(D).reshape(D // 32, 2, 16).transpose(0, 2, 1).reshape(-1)


def _sc_gather(ids_flat, table_pairs):
    mesh = plsc.VectorSubcoreMesh(core_axis_name="c", subcore_axis_name="s")

    @functools.partial(
        pl.kernel,
        mesh=mesh,
        compiler_params=pltpu.CompilerParams(
            use_tc_tiling_on_sc=False, needs_layout_passes=False
        ),
        out_type=jax.ShapeDtypeStruct((B_TOT * D,), jnp.float32),
        scratch_types=[
            pltpu.VMEM((B_PER_W,), jnp.int32),
            pltpu.VMEM((C, PAIRS), jnp.int32),
            pltpu.VMEM((C, PAIRS), jnp.int32),
            pltpu.VMEM((C * D,), jnp.float32),
            pltpu.VMEM((C * D,), jnp.float32),
            pltpu.SemaphoreType.DMA,
            pltpu.SemaphoreType.DMA,
            pltpu.SemaphoreType.DMA,
            pltpu.SemaphoreType.DMA,
        ],
    )
    def k(ids_hbm, tbl_hbm, out_hbm, idx_v, bf0, bf1, ob0, ob1, g0, g1, s0, s1):
        wid = lax.axis_index("s") * NC + lax.axis_index("c")
        base = pl.multiple_of(wid * B_PER_W, 8)
        pltpu.sync_copy(ids_hbm.at[pl.ds(base, B_PER_W)], idx_v)

        bfs = (bf0, bf1)
        obs = (ob0, ob1)
        gsems = (g0, g1)
        ssems = (s0, s1)
        HIMASK = jnp.int32(-65536)

        def g_start(chunk, b):
            off = pl.multiple_of(chunk * C, 8)
            pltpu.async_copy(
                tbl_hbm.at[idx_v.at[pl.ds(off, C)]], bfs[b], gsems[b]
            )

        def g_wait(b):
            pltpu.make_async_copy(
                tbl_hbm.at[pl.ds(0, C)], bfs[b], gsems[b]
            ).wait()

        def s_start(chunk, b):
            flat = pl.multiple_of((base + chunk * C) * D, 8)
            pltpu.async_copy(
                obs[b], out_hbm.at[pl.ds(flat, C * D)], ssems[b]
            )

        def s_wait(b):
            pltpu.make_async_copy(
                out_hbm.at[pl.ds(0, C * D)], obs[b], ssems[b]
            ).wait()

        def widen(b):
            def row_body(r, carry):
                dst = pl.multiple_of(r * D, 8)
                prev = None
                for v in range(PAIRS // 16):
                    x = bfs[b][r, pl.ds(v * 16, 16)]
                    if prev is not None:
                        pv, px = prev
                        obs[b][pl.ds(dst + pv * 32, 16)] = plsc.bitcast(
                            px << 16, jnp.float32
                        )
                        obs[b][pl.ds(dst + pv * 32 + 16, 16)] = plsc.bitcast(
                            px & HIMASK, jnp.float32
                        )
                    prev = (v, x)
                pv, px = prev
                obs[b][pl.ds(dst + pv * 32, 16)] = plsc.bitcast(
                    px << 16, jnp.float32
                )
                obs[b][pl.ds(dst + pv * 32 + 16, 16)] = plsc.bitcast(
                    px & HIMASK, jnp.float32
                )
                return carry

            lax.fori_loop(0, C, row_body, 0)

        g_start(0, 0)
        g_start(1, 1)

        def pair(p2, carry):
            for b in range(2):
                chunk = p2 * 2 + b
                g_wait(b)

                @pl.when(chunk >= 2)
                def _():
                    s_wait(b)

                widen(b)
                s_start(chunk, b)

                @pl.when(chunk + 2 < NCHUNK)
                def _():
                    g_start(chunk + 2, b)

            return carry

        lax.fori_loop(0, NCHUNK // 2, pair, 0)
        s_wait(0)
        s_wait(1)

    return k(ids_flat, table_pairs)


def kernel(phoneme_ids, table):
    ids_flat = phoneme_ids.reshape(-1)
    tbl_bf = table[:, _PERM].astype(jnp.bfloat16)
    table_pairs = lax.bitcast_convert_type(
        tbl_bf.reshape(VOCAB, PAIRS, 2), jnp.int32
    )
    out = _sc_gather(ids_flat, table_pairs)
    return out.reshape(phoneme_ids.shape[0], phoneme_ids.shape[1], D)


# speedup vs baseline: 1.2004x; 1.0031x over previous
import functools

import jax
import jax.numpy as jnp
import numpy as np
from jax import lax
from jax.experimental import pallas as pl
from jax.experimental.pallas import tpu as pltpu
from jax.experimental.pallas import tpu_sc as plsc

D = 1024
VOCAB = 194
PAIRS = D // 2
NC = 2
NS = 16
NW = NC * NS
B_TOT = 1024 * 200
B_PER_W = B_TOT // NW
C = 40
NCHUNK = B_PER_W // C

_PERM = np.arange(D).reshape(D // 32, 2, 16).transpose(0, 2, 1).reshape(-1)


def _sc_gather(ids_flat, table_pairs):
    mesh = plsc.VectorSubcoreMesh(core_axis_name="c", subcore_axis_name="s")

    @functools.partial(
        pl.kernel,
        mesh=mesh,
        compiler_params=pltpu.CompilerParams(
            use_tc_tiling_on_sc=False, needs_layout_passes=False
        ),
        out_type=jax.ShapeDtypeStruct((B_TOT * D,), jnp.float32),
        scratch_types=[
            pltpu.VMEM((B_PER_W,), jnp.int32),
            pltpu.VMEM((C, PAIRS), jnp.int32),
            pltpu.VMEM((C, PAIRS), jnp.int32),
            pltpu.VMEM((C * D,), jnp.float32),
            pltpu.VMEM((C * D,), jnp.float32),
            pltpu.SemaphoreType.DMA,
            pltpu.SemaphoreType.DMA,
            pltpu.SemaphoreType.DMA,
            pltpu.SemaphoreType.DMA,
        ],
    )
    def k(ids_hbm, tbl_hbm, out_hbm, idx_v, bf0, bf1, ob0, ob1, g0, g1, s0, s1):
        wid = lax.axis_index("s") * NC + lax.axis_index("c")
        base = pl.multiple_of(wid * B_PER_W, 8)
        pltpu.sync_copy(ids_hbm.at[pl.ds(base, B_PER_W)], idx_v)

        bfs = (bf0, bf1)
        obs = (ob0, ob1)
        gsems = (g0, g1)
        ssems = (s0, s1)
        HIMASK = jnp.int32(-65536)

        def g_start(chunk, b):
            off = pl.multiple_of(chunk * C, 8)
            pltpu.async_copy(
                tbl_hbm.at[idx_v.at[pl.ds(off, C)]], bfs[b], gsems[b]
            )

        def g_wait(b):
            pltpu.make_async_copy(
                tbl_hbm.at[pl.ds(0, C)], bfs[b], gsems[b]
            ).wait()

        def s_start(chunk, b):
            flat = pl.multiple_of((base + chunk * C) * D, 8)
            pltpu.async_copy(
                obs[b], out_hbm.at[pl.ds(flat, C * D)], ssems[b]
            )

        def s_wait(b):
            pltpu.make_async_copy(
                out_hbm.at[pl.ds(0, C * D)], obs[b], ssems[b]
            ).wait()

        def widen(b):
            def flush(dst0, dst1, prev):
                pv, xa, xb = prev
                obs[b][pl.ds(dst0 + pv * 32, 16)] = plsc.bitcast(
                    xa << 16, jnp.float32
                )
                obs[b][pl.ds(dst0 + pv * 32 + 16, 16)] = plsc.bitcast(
                    xa & HIMASK, jnp.float32
                )
                obs[b][pl.ds(dst1 + pv * 32, 16)] = plsc.bitcast(
                    xb << 16, jnp.float32
                )
                obs[b][pl.ds(dst1 + pv * 32 + 16, 16)] = plsc.bitcast(
                    xb & HIMASK, jnp.float32
                )

            def row_body(rp, carry):
                r0 = rp * 2
                dst0 = pl.multiple_of(r0 * D, 8)
                dst1 = pl.multiple_of(r0 * D + D, 8)
                prev = None
                for v in range(PAIRS // 16):
                    xa = bfs[b][r0, pl.ds(v * 16, 16)]
                    xb = bfs[b][r0 + 1, pl.ds(v * 16, 16)]
                    if prev is not None:
                        flush(dst0, dst1, prev)
                    prev = (v, xa, xb)
                flush(dst0, dst1, prev)
                return carry

            lax.fori_loop(0, C // 2, row_body, 0)

        g_start(0, 0)
        g_start(1, 1)

        def pair(p2, carry):
            for b in range(2):
                chunk = p2 * 2 + b
                g_wait(b)

                @pl.when(chunk >= 2)
                def _():
                    s_wait(b)

                widen(b)
                s_start(chunk, b)

                @pl.when(chunk + 2 < NCHUNK)
                def _():
                    g_start(chunk + 2, b)

            return carry

        lax.fori_loop(0, NCHUNK // 2, pair, 0)
        s_wait(0)
        s_wait(1)

    return k(ids_flat, table_pairs)


def kernel(phoneme_ids, table):
    ids_flat = phoneme_ids.reshape(-1)
    tbl_bf = table[:, _PERM].astype(jnp.bfloat16)
    table_pairs = lax.bitcast_convert_type(
        tbl_bf.reshape(VOCAB, PAIRS, 2), jnp.int32
    )
    out = _sc_gather(ids_flat, table_pairs)
    return out.reshape(phoneme_ids.shape[0], phoneme_ids.shape[1], D)


# speedup vs baseline: 2.0400x; 1.6995x over previous
import functools

import jax
import jax.numpy as jnp
from jax import lax
from jax.experimental import pallas as pl
from jax.experimental.pallas import tpu as pltpu
from jax.experimental.pallas import tpu_sc as plsc

D = 1024
NC = 2
NS = 16
NW = NC * NS
B_TOT = 1024 * 200
B_PER_W = B_TOT // NW
C = 40
NCHUNK = B_PER_W // C


def _sc_gather(ids_flat, table):
    mesh = plsc.VectorSubcoreMesh(core_axis_name="c", subcore_axis_name="s")

    @functools.partial(
        pl.kernel,
        mesh=mesh,
        out_type=jax.ShapeDtypeStruct((B_TOT, D), jnp.float32),
        scratch_types=[
            pltpu.VMEM((B_PER_W,), jnp.int32),
            pltpu.VMEM((C, D), jnp.float32),
            pltpu.VMEM((C, D), jnp.float32),
            pltpu.SemaphoreType.DMA,
            pltpu.SemaphoreType.DMA,
            pltpu.SemaphoreType.DMA,
            pltpu.SemaphoreType.DMA,
        ],
    )
    def k(ids_hbm, table_hbm, out_hbm, idx_v, buf0, buf1, g0, g1, s0, s1):
        wid = lax.axis_index("s") * NC + lax.axis_index("c")
        base = pl.multiple_of(wid * B_PER_W, 8)
        pltpu.sync_copy(ids_hbm.at[pl.ds(base, B_PER_W)], idx_v)

        bufs = (buf0, buf1)
        gsems = (g0, g1)
        ssems = (s0, s1)

        def g_start(chunk, buf, sem):
            off = pl.multiple_of(chunk * C, 8)
            pltpu.async_copy(table_hbm.at[idx_v.at[pl.ds(off, C)]], buf, sem)

        def g_wait(buf, sem):
            pltpu.make_async_copy(table_hbm.at[pl.ds(0, C)], buf, sem).wait()

        def s_start(chunk, buf, sem):
            row = pl.multiple_of(base + chunk * C, 8)
            return pltpu.async_copy(buf, out_hbm.at[pl.ds(row, C)], sem)

        g_start(0, buf0, g0)
        g_start(1, buf1, g1)

        def pair(p, carry):
            for b in range(2):
                chunk = p * 2 + b
                g_wait(bufs[b], gsems[b])
                cp = s_start(chunk, bufs[b], ssems[b])
                cp.wait()

                @pl.when(chunk + 2 < NCHUNK)
                def _():
                    g_start(chunk + 2, bufs[b], gsems[b])

            return carry

        lax.fori_loop(0, NCHUNK // 2, pair, 0)

    return k(ids_flat, table)


def kernel(phoneme_ids, table):
    ids_flat = phoneme_ids.reshape(-1)
    out = _sc_gather(ids_flat, table)
    return out.reshape(phoneme_ids.shape[0], phoneme_ids.shape[1], D)
